# baseline (speedup 1.0000x reference)
import jax
import jax.numpy as jnp
from jax import lax
from jax.experimental import pallas as pl
from jax.experimental.pallas import tpu as pltpu
from jax.experimental.pallas import tpu_sc as plsc

_D = 64
_B = 16384
_DIM = 128

_NC = 2
_NS = 16
_NW = _NC * _NS
_BPW = _B // _NW
_CHUNK = 128
_NCHUNK = _BPW // _CHUNK
_HALF = 2


def _gather_body(x_hbm, e1_hbm, e2_hbm, e3_hbm, o1, o2, o3, idx_v, rows_v, sem):
    wid = lax.axis_index("s") * _NC + lax.axis_index("c")
    base = wid * _BPW
    cbase = wid * _NCHUNK
    tables = (e1_hbm, e2_hbm, e3_hbm)
    outs = (o1, o2, o3)
    for t in range(3):
        pltpu.sync_copy(x_hbm.at[t, pl.ds(cbase, _NCHUNK)], idx_v.at[t])
    for h in range(_NCHUNK // _HALF):
        copies = []
        for t in range(3):
            for j in range(_HALF):
                copies.append(
                    pltpu.async_copy(
                        tables[t].at[idx_v.at[t, h * _HALF + j]],
                        rows_v.at[t, pl.ds(j * _CHUNK, _CHUNK)],
                        sem,
                    )
                )
        for c in copies:
            c.wait()
        for t in range(3):
            pltpu.sync_copy(
                rows_v.at[t],
                outs[t].at[pl.ds(base + h * _HALF * _CHUNK, _HALF * _CHUNK)],
            )


def _make_gather():
    mesh = plsc.VectorSubcoreMesh(core_axis_name="c", subcore_axis_name="s")
    out = jax.ShapeDtypeStruct((_B, 2 * _D), jnp.float32)
    return pl.kernel(
        _gather_body,
        mesh=mesh,
        out_type=(out, out, out),
        scratch_types=[
            pltpu.VMEM((3, _NCHUNK, _CHUNK), jnp.int32),
            pltpu.VMEM((3, _HALF * _CHUNK, 2 * _D), jnp.float32),
            pltpu.SemaphoreType.DMA,
        ],
        compiler_params=pltpu.CompilerParams(use_tc_tiling_on_sc=True),
    )


def _mlp_body(p1, p2, p3, par, w1, b1, w2, b2, o):
    h = None
    for t, pt in enumerate((p1, p2, p3)):
        x = pt[...]
        pcol = par[:, t:t + 1]
        sel = x[:, :_D] * (1.0 - pcol) + x[:, _D:] * pcol
        acc = jnp.dot(sel, w1[t * _D:(t + 1) * _D, :],
                      preferred_element_type=jnp.float32)
        h = acc if h is None else h + acc
    h = h + b1[...]
    h = h * jax.nn.sigmoid(h)
    o[...] = jnp.dot(h, w2[...], preferred_element_type=jnp.float32) + b2[...]


def _mlp_call(p1, p2, p3, par, W1, b1, W2, b2):
    r = 2048
    espec = pl.BlockSpec((r, 2 * _D), lambda i: (i, 0))
    full = lambda s: pl.BlockSpec(s, lambda i: (0, 0))
    return pl.pallas_call(
        _mlp_body,
        grid=(_B // r,),
        in_specs=[espec, espec, espec,
                  pl.BlockSpec((r, 3), lambda i: (i, 0)),
                  full((3 * _D, _DIM)), full((1, _DIM)),
                  full((_DIM, _DIM)), full((1, _DIM))],
        out_specs=pl.BlockSpec((r, _DIM), lambda i: (i, 0)),
        out_shape=jax.ShapeDtypeStruct((_B, _DIM), jnp.float32),
    )(p1, p2, p3, par, W1, b1.reshape(1, _DIM), W2, b2.reshape(1, _DIM))


def _pack_body(l1, h1, l2, h2, l3, h3, o1, o2, o3):
    for lo, hi, o in ((l1, h1, o1), (l2, h2, o2), (l3, h3, o3)):
        o[:, :_D] = lo[...]
        o[:, _D:] = hi[...]


def _pack_call(E1, E2, E3):
    rb = 5000
    nh = E1.shape[0] // 2
    lospec = pl.BlockSpec((rb, _D), lambda i: (i, 0))
    hispec = pl.BlockSpec((rb, _D), lambda i, nb=nh // rb: (i + nb, 0))
    outspec = pl.BlockSpec((rb, 2 * _D), lambda i: (i, 0))
    oshape = jax.ShapeDtypeStruct((nh, 2 * _D), jnp.float32)
    return pl.pallas_call(
        _pack_body,
        grid=(nh // rb,),
        in_specs=[lospec, hispec, lospec, hispec, lospec, hispec],
        out_specs=(outspec, outspec, outspec),
        out_shape=(oshape, oshape, oshape),
    )(E1, E1, E2, E2, E3, E3)


def kernel(x, E1, E2, E3, W1, b1, W2, b2):
    xi = x.astype(jnp.int32)
    nh = E1.shape[0] // 2
    hi = (xi >= nh)
    x_r = (xi - jnp.where(hi, nh, 0)).reshape(3, _B // _CHUNK, _CHUNK)
    par = hi.astype(jnp.float32).T
    gather = _make_gather()
    E1p, E2p, E3p = _pack_call(E1, E2, E3)
    p1, p2, p3 = gather(x_r, E1p, E2p, E3p)
    return _mlp_call(p1, p2, p3, par, W1, b1, W2, b2)


# speedup vs baseline: 1.1772x; 1.1772x over previous
import jax
import jax.numpy as jnp
from jax import lax
from jax.experimental import pallas as pl
from jax.experimental.pallas import tpu as pltpu
from jax.experimental.pallas import tpu_sc as plsc

_D = 64
_B = 16384
_DIM = 128

_NC = 2
_NS = 16
_NW = _NC * _NS
_BPW = _B // _NW
_CHUNK = 128
_NCHUNK = _BPW // _CHUNK
_HALF = 2


def _gather_body(x_hbm, e_hbm, o, idx_v, rows_v, sem):
    wid = lax.axis_index("s") * _NC + lax.axis_index("c")
    base = wid * _BPW
    cbase = wid * _NCHUNK
    pltpu.sync_copy(x_hbm.at[pl.ds(cbase, _NCHUNK)], idx_v)
    for h in range(_NCHUNK // _HALF):
        copies = [
            pltpu.async_copy(
                e_hbm.at[idx_v.at[h * _HALF + j]],
                rows_v.at[pl.ds(j * _CHUNK, _CHUNK)],
                sem,
            )
            for j in range(_HALF)
        ]
        for c in copies:
            c.wait()
        pltpu.sync_copy(
            rows_v,
            o.at[pl.ds(base + h * _HALF * _CHUNK, _HALF * _CHUNK)],
        )


def _make_gather():
    mesh = plsc.VectorSubcoreMesh(core_axis_name="c", subcore_axis_name="s")
    return pl.kernel(
        _gather_body,
        mesh=mesh,
        out_type=jax.ShapeDtypeStruct((_B, 2 * _D), jnp.float32),
        scratch_types=[
            pltpu.VMEM((_NCHUNK, _CHUNK), jnp.int32),
            pltpu.VMEM((_HALF * _CHUNK, 2 * _D), jnp.float32),
            pltpu.SemaphoreType.DMA,
        ],
        compiler_params=pltpu.CompilerParams(use_tc_tiling_on_sc=True),
    )


def _mlp_body(p1, p2, p3, par, w1, b1, w2, b2, o):
    h = None
    for t, pt in enumerate((p1, p2, p3)):
        x = pt[...]
        pcol = par[:, t:t + 1]
        sel = x[:, :_D] * (1.0 - pcol) + x[:, _D:] * pcol
        acc = jnp.dot(sel, w1[t * _D:(t + 1) * _D, :],
                      preferred_element_type=jnp.float32)
        h = acc if h is None else h + acc
    h = h + b1[...]
    h = h * jax.nn.sigmoid(h)
    o[...] = jnp.dot(h, w2[...], preferred_element_type=jnp.float32) + b2[...]


def _mlp_call(p1, p2, p3, par, W1, b1, W2, b2):
    r = 2048
    espec = pl.BlockSpec((r, 2 * _D), lambda i: (i, 0))
    full = lambda s: pl.BlockSpec(s, lambda i: (0, 0))
    return pl.pallas_call(
        _mlp_body,
        grid=(_B // r,),
        in_specs=[espec, espec, espec,
                  pl.BlockSpec((r, 3), lambda i: (i, 0)),
                  full((3 * _D, _DIM)), full((1, _DIM)),
                  full((_DIM, _DIM)), full((1, _DIM))],
        out_specs=pl.BlockSpec((r, _DIM), lambda i: (i, 0)),
        out_shape=jax.ShapeDtypeStruct((_B, _DIM), jnp.float32),
    )(p1, p2, p3, par, W1, b1.reshape(1, _DIM), W2, b2.reshape(1, _DIM))


def kernel(x, E1, E2, E3, W1, b1, W2, b2):
    xi = x.astype(jnp.int32)
    x_r = (xi >> 1).reshape(3, _B // _CHUNK, _CHUNK)
    par = (xi & 1).astype(jnp.float32).T
    gather = _make_gather()
    ps = [
        gather(x_r[t], E.reshape(E.shape[0] // 2, 2 * _D))
        for t, E in enumerate((E1, E2, E3))
    ]
    return _mlp_call(ps[0], ps[1], ps[2], par, W1, b1, W2, b2)
